# baseline (speedup 1.0000x reference)
import functools

import jax
import jax.numpy as jnp
from jax import lax
from jax.experimental import pallas as pl
from jax.experimental.pallas import tpu as pltpu
from jax.experimental.pallas import tpu_sc as plsc

NC = 2
NS = 16
NW = NC * NS
LANES = 16

CHUNK = 128
PC = 64


def _enc_gather_body(n_per_w, n_ctx_chunks, n_live_chunks, d,
                     table_hbm, ids_hbm, ctx_out, sf_out,
                     idx_v, rows0_v, rows1_v, sem0, sem1):
    nk = n_per_w // CHUNK
    wid = lax.axis_index("s") * NC + lax.axis_index("c")
    base = wid * n_per_w
    pltpu.sync_copy(ids_hbm.at[pl.ds(base, n_per_w)], idx_v)
    bufs = (rows0_v, rows1_v)
    sems = (sem0, sem1)
    handles = [None, None]
    handles[0] = pltpu.async_copy(
        table_hbm.at[idx_v.at[pl.ds(0, CHUNK)]], bufs[0], sems[0])
    for k in range(nk):
        handles[k % 2].wait()
        if k + 1 < nk:
            handles[(k + 1) % 2] = pltpu.async_copy(
                table_hbm.at[idx_v.at[pl.ds((k + 1) * CHUNK, CHUNK)]],
                bufs[(k + 1) % 2], sems[(k + 1) % 2])
        j = wid * nk + k

        @pl.when(j < n_ctx_chunks)
        def _():
            pltpu.sync_copy(bufs[k % 2], ctx_out.at[pl.ds(j * CHUNK, CHUNK)])

        @pl.when(jnp.logical_and(j >= n_ctx_chunks, j < n_live_chunks))
        def _():
            pltpu.sync_copy(
                bufs[k % 2],
                sf_out.at[pl.ds((j - n_ctx_chunks) * CHUNK, CHUNK)])


def _make_enc_gather(n_ids, n_ctx, n_sf, v, d):
    n_per_w = n_ids // NW
    mesh = plsc.VectorSubcoreMesh(core_axis_name="c", subcore_axis_name="s")
    return pl.kernel(
        functools.partial(_enc_gather_body, n_per_w, n_ctx // CHUNK,
                          (n_ctx + n_sf) // CHUNK, d),
        out_type=(
            jax.ShapeDtypeStruct((n_ctx, d), jnp.float32),
            jax.ShapeDtypeStruct((n_sf, d), jnp.float32),
        ),
        mesh=mesh,
        scratch_types=[
            pltpu.VMEM((n_per_w,), jnp.int32),
            pltpu.VMEM((CHUNK, d), jnp.float32),
            pltpu.VMEM((CHUNK, d), jnp.float32),
            pltpu.SemaphoreType.DMA,
            pltpu.SemaphoreType.DMA,
        ],
    )


def _lf_gather_body(n_pairs, p_per_w, l, d, mu_hbm, ls_hbm, idst_hbm,
                    mu_out, sig_out, *scr):
    idx_vs = scr[:l]
    s_vs = scr[l:2 * l]
    rows = (scr[2 * l:3 * l], scr[3 * l:4 * l])
    acc_v, sig_v, sem_s, sem_g0, sem_g1 = scr[4 * l:]
    sems = (sem_g0, sem_g1)
    nk = p_per_w // PC
    wid = lax.axis_index("s") * NC + lax.axis_index("c")
    base_p = wid * p_per_w
    for j in range(l):
        pltpu.sync_copy(idst_hbm.at[pl.ds(j * n_pairs + base_p, p_per_w)],
                        idx_vs[j])

    sig_handles = []
    for j in range(l):
        for k in range(p_per_w // CHUNK):
            sig_handles.append(pltpu.async_copy(
                ls_hbm.at[idx_vs[j].at[pl.ds(k * CHUNK, CHUNK)]],
                s_vs[j].at[pl.ds(k * CHUNK, CHUNK)], sem_s))

    def fire_mu(k):
        par = k % 2
        return [pltpu.async_copy(
            mu_hbm.at[idx_vs[j].at[pl.ds(k * PC, PC)]], rows[par][j],
            sems[par]) for j in range(l)]

    mu_handles = [None, None]
    mu_handles[0] = fire_mu(0)

    for h in sig_handles:
        h.wait()

    def sig_group(g, _):
        sl = pl.ds(g * LANES, LANES)
        acc = jnp.exp(s_vs[0][sl])
        for j in range(1, l):
            acc = acc + jnp.exp(s_vs[j][sl])
        sig_v[sl] = acc
        return 0

    lax.fori_loop(0, p_per_w // LANES, sig_group, 0)
    pltpu.sync_copy(sig_v, sig_out.at[pl.ds(base_p, p_per_w)])

    for k in range(nk):
        par = k % 2
        for h in mu_handles[par]:
            h.wait()
        if k + 1 < nk:
            mu_handles[(k + 1) % 2] = fire_mu(k + 1)
        bufs = rows[par]

        def pair_red(p, _):
            for g in range(d // LANES):
                sl = pl.ds(g * LANES, LANES)
                acc = bufs[0][p, sl]
                for j in range(1, l):
                    acc = acc + bufs[j][p, sl]
                acc_v[p, sl] = acc
            return 0

        lax.fori_loop(0, PC, pair_red, 0)
        pltpu.sync_copy(acc_v, mu_out.at[pl.ds(base_p + k * PC, PC)])


def _make_lf_gather(n_pairs, l, v, d):
    p_per_w = n_pairs // NW
    mesh = plsc.VectorSubcoreMesh(core_axis_name="c", subcore_axis_name="s")
    return pl.kernel(
        functools.partial(_lf_gather_body, n_pairs, p_per_w, l, d),
        out_type=(
            jax.ShapeDtypeStruct((n_pairs, d), jnp.float32),
            jax.ShapeDtypeStruct((n_pairs,), jnp.float32),
        ),
        mesh=mesh,
        scratch_types=(
            [pltpu.VMEM((p_per_w,), jnp.int32) for _ in range(l)]
            + [pltpu.VMEM((p_per_w,), jnp.float32) for _ in range(l)]
            + [pltpu.VMEM((PC, d), jnp.float32) for _ in range(2 * l)]
            + [
                pltpu.VMEM((PC, d), jnp.float32),
                pltpu.VMEM((p_per_w,), jnp.float32),
                pltpu.SemaphoreType.DMA,
                pltpu.SemaphoreType.DMA,
                pltpu.SemaphoreType.DMA,
            ]
        ),
    )


def _dense_body(c, m, d, center_ref, ctx_ref, mu_ref, sig_ref, tct_ref,
                numo_ref, wc_ref, wx_ref, fb_ref, uw_ref, ub_ref, vw_ref,
                vb_ref, score_ref, sigq_ref):
    bb = center_ref.shape[0]
    f32 = jnp.float32
    ctx = ctx_ref[...]
    hx = lax.dot_general(ctx, wx_ref[...], (((2,), (1,)), ((), ())),
                         preferred_element_type=f32)
    hc = lax.dot_general(center_ref[...], wc_ref[...], (((1,), (1,)), ((), ())),
                         preferred_element_type=f32)
    hr = jnp.maximum(hx + hc[None, :, :] + fb_ref[...][None, :, :], 0.0)
    h = jnp.mean(hr, axis=0)
    mu_q = lax.dot_general(h, uw_ref[...], (((1,), (1,)), ((), ())),
                           preferred_element_type=f32) + ub_ref[...]
    lsq = jnp.sum(h * vw_ref[...], axis=1) + vb_ref[0, 0]
    var_q = jnp.exp(2.0 * lsq)

    norm = jnp.maximum(tct_ref[...], 1.0)
    inv_n = 1.0 / norm
    ns = mu_ref[...]
    ones_d = jnp.full((d, 1), 1.0, dtype=f32)
    q2 = lax.dot_general(mu_q * mu_q, ones_d, (((1,), (0,)), ((), ())),
                         preferred_element_type=f32)
    p2 = lax.dot_general(ns * ns, ones_d, (((2,), (0,)), ((), ())),
                         preferred_element_type=f32)[..., 0]
    t3 = ns * mu_q[:, None, :]
    cross = lax.dot_general(t3, ones_d, (((2,), (0,)), ((), ())),
                            preferred_element_type=f32)[..., 0]
    sq = q2 - 2.0 * cross * inv_n + p2 * inv_n * inv_n
    sig_p = sig_ref[...] * inv_n
    kl = (d * (jnp.log(sig_p) - lsq[:, None])
          + (d * var_q[:, None] + sq) / (2.0 * sig_p * sig_p)
          - 0.5 * d)
    mrange = lax.broadcasted_iota(jnp.int32, (bb, m), 1).astype(f32)
    score = jnp.where(mrange >= numo_ref[...], -jnp.inf, -kl)
    score_ref[...] = score
    sigq_ref[...] = jnp.exp(lsq)[:, None]


def _make_dense(b, c, m, d, bb):
    grid = (b // bb,)
    full = lambda i: (0, 0)
    return pl.pallas_call(
        functools.partial(_dense_body, c, m, d),
        grid=grid,
        in_specs=[
            pl.BlockSpec((bb, d), lambda i: (i, 0)),
            pl.BlockSpec((c, bb, d), lambda i: (0, i, 0)),
            pl.BlockSpec((bb, m, d), lambda i: (i, 0, 0)),
            pl.BlockSpec((bb, m), lambda i: (i, 0)),
            pl.BlockSpec((bb, m), lambda i: (i, 0)),
            pl.BlockSpec((bb, 1), lambda i: (i, 0)),
            pl.BlockSpec((d, d), full),
            pl.BlockSpec((d, d), full),
            pl.BlockSpec((1, d), full),
            pl.BlockSpec((d, d), full),
            pl.BlockSpec((1, d), full),
            pl.BlockSpec((1, d), full),
            pl.BlockSpec((1, 1), full),
        ],
        out_specs=[
            pl.BlockSpec((bb, m), lambda i: (i, 0)),
            pl.BlockSpec((bb, 1), lambda i: (i, 0)),
        ],
        out_shape=[
            jax.ShapeDtypeStruct((b, m), jnp.float32),
            jax.ShapeDtypeStruct((b, 1), jnp.float32),
        ],
    )


def kernel(sf_ids, context_ids, lf_ids, target_lf_ids, lf_token_ct,
           num_outputs, emb_mu, emb_log_sigma, enc_emb, f_w, f_b, u_w, u_b,
           v_w, v_b):
    b, c = context_ids.shape
    m, l = lf_ids.shape[1], lf_ids.shape[2]
    v, d = emb_mu.shape

    n_ids = b * (c + 1)
    gran = NW * CHUNK
    n_pad = -(-n_ids // gran) * gran
    all_ids = jnp.concatenate(
        [context_ids.T.reshape(-1), sf_ids,
         jnp.zeros((n_pad - n_ids,), jnp.int32)])
    ids_t = lf_ids.reshape(b * m, l).T.reshape(-1)
    mu_sum, sig_sum = _make_lf_gather(b * m, l, v, d)(
        emb_mu, emb_log_sigma.reshape(-1), ids_t)
    ctx2, center = _make_enc_gather(n_pad, b * c, b, v, d)(enc_emb, all_ids)
    ctx = ctx2.reshape(c, b, d)
    mu_sum = mu_sum.reshape(b, m, d)
    sig_sum = sig_sum.reshape(b, m)

    wc = f_w[:, :d]
    wx = f_w[:, d:]
    score, sigq = _make_dense(b, c, m, d, 64)(
        center, ctx, mu_sum, sig_sum, lf_token_ct,
        num_outputs.astype(jnp.float32)[:, None],
        wc, wx, f_b[None, :], u_w, u_b[None, :], v_w, v_b[:, None])
    return score, target_lf_ids, sigq


# speedup vs baseline: 1.0753x; 1.0753x over previous
import functools

import jax
import jax.numpy as jnp
from jax import lax
from jax.experimental import pallas as pl
from jax.experimental.pallas import tpu as pltpu
from jax.experimental.pallas import tpu_sc as plsc

NC = 2
NS = 16
NW = NC * NS
LANES = 16

CHUNK = 128
PC = 64


def _enc_gather_body(n_per_w, n_ctx_chunks, n_live_chunks, d,
                     table_hbm, ids_hbm, ctx_out, sf_out,
                     idx_v, rows0_v, rows1_v, sem0, sem1):
    nk = n_per_w // CHUNK
    wid = lax.axis_index("s") * NC + lax.axis_index("c")
    base = wid * n_per_w
    pltpu.sync_copy(ids_hbm.at[pl.ds(base, n_per_w)], idx_v)
    bufs = (rows0_v, rows1_v)
    sems = (sem0, sem1)
    handles = [None, None]
    handles[0] = pltpu.async_copy(
        table_hbm.at[idx_v.at[pl.ds(0, CHUNK)]], bufs[0], sems[0])
    for k in range(nk):
        handles[k % 2].wait()
        if k + 1 < nk:
            handles[(k + 1) % 2] = pltpu.async_copy(
                table_hbm.at[idx_v.at[pl.ds((k + 1) * CHUNK, CHUNK)]],
                bufs[(k + 1) % 2], sems[(k + 1) % 2])
        j = wid * nk + k

        @pl.when(j < n_ctx_chunks)
        def _():
            pltpu.sync_copy(bufs[k % 2], ctx_out.at[pl.ds(j * CHUNK, CHUNK)])

        @pl.when(jnp.logical_and(j >= n_ctx_chunks, j < n_live_chunks))
        def _():
            pltpu.sync_copy(
                bufs[k % 2],
                sf_out.at[pl.ds((j - n_ctx_chunks) * CHUNK, CHUNK)])


def _make_enc_gather(n_ids, n_ctx, n_sf, v, d):
    n_per_w = n_ids // NW
    mesh = plsc.VectorSubcoreMesh(core_axis_name="c", subcore_axis_name="s")
    return pl.kernel(
        functools.partial(_enc_gather_body, n_per_w, n_ctx // CHUNK,
                          (n_ctx + n_sf) // CHUNK, d),
        out_type=(
            jax.ShapeDtypeStruct((n_ctx, d), jnp.float32),
            jax.ShapeDtypeStruct((n_sf, d), jnp.float32),
        ),
        mesh=mesh,
        scratch_types=[
            pltpu.VMEM((n_per_w,), jnp.int32),
            pltpu.VMEM((CHUNK, d), jnp.float32),
            pltpu.VMEM((CHUNK, d), jnp.float32),
            pltpu.SemaphoreType.DMA,
            pltpu.SemaphoreType.DMA,
        ],
    )


def _lf_gather_body(n_pairs, p_per_w, l, d, mu_hbm, ls_hbm, idst_hbm,
                    mu_out, sig_out, *scr):
    idx_vs = scr[:l]
    s_vs = scr[l:2 * l]
    rows = (scr[2 * l:3 * l], scr[3 * l:4 * l])
    acc_v, sig_v, sem_s, sem_g0, sem_g1 = scr[4 * l:]
    sems = (sem_g0, sem_g1)
    nk = p_per_w // PC
    wid = lax.axis_index("s") * NC + lax.axis_index("c")
    base_p = wid * p_per_w
    for j in range(l):
        pltpu.sync_copy(idst_hbm.at[pl.ds(j * n_pairs + base_p, p_per_w)],
                        idx_vs[j])

    sig_handles = []
    for j in range(l):
        for k in range(p_per_w // CHUNK):
            sig_handles.append(pltpu.async_copy(
                ls_hbm.at[idx_vs[j].at[pl.ds(k * CHUNK, CHUNK)]],
                s_vs[j].at[pl.ds(k * CHUNK, CHUNK)], sem_s))

    def fire_mu(k):
        par = k % 2
        return [pltpu.async_copy(
            mu_hbm.at[idx_vs[j].at[pl.ds(k * PC, PC)]], rows[par][j],
            sems[par]) for j in range(l)]

    mu_handles = [None, None]
    mu_handles[0] = fire_mu(0)

    for h in sig_handles:
        h.wait()

    def sig_group(g, _):
        sl = pl.ds(g * LANES, LANES)
        acc = jnp.exp(s_vs[0][sl])
        for j in range(1, l):
            acc = acc + jnp.exp(s_vs[j][sl])
        sig_v[sl] = acc
        return 0

    lax.fori_loop(0, p_per_w // LANES, sig_group, 0)
    pltpu.sync_copy(sig_v, sig_out.at[pl.ds(base_p, p_per_w)])

    for k in range(nk):
        par = k % 2
        for h in mu_handles[par]:
            h.wait()
        if k + 1 < nk:
            mu_handles[(k + 1) % 2] = fire_mu(k + 1)
        bufs = rows[par]

        def pair_red(p, _):
            for g in range(d // LANES):
                sl = pl.ds(g * LANES, LANES)
                acc = bufs[0][p, sl]
                for j in range(1, l):
                    acc = acc + bufs[j][p, sl]
                acc_v[p, sl] = acc
            return 0

        lax.fori_loop(0, PC, pair_red, 0)
        pltpu.sync_copy(acc_v, mu_out.at[pl.ds(base_p + k * PC, PC)])


def _make_lf_gather(n_pairs, l, v, d):
    p_per_w = n_pairs // NW
    mesh = plsc.VectorSubcoreMesh(core_axis_name="c", subcore_axis_name="s")
    return pl.kernel(
        functools.partial(_lf_gather_body, n_pairs, p_per_w, l, d),
        out_type=(
            jax.ShapeDtypeStruct((n_pairs, d), jnp.float32),
            jax.ShapeDtypeStruct((n_pairs,), jnp.float32),
        ),
        mesh=mesh,
        scratch_types=(
            [pltpu.VMEM((p_per_w,), jnp.int32) for _ in range(l)]
            + [pltpu.VMEM((p_per_w,), jnp.float32) for _ in range(l)]
            + [pltpu.VMEM((PC, d), jnp.float32) for _ in range(2 * l)]
            + [
                pltpu.VMEM((PC, d), jnp.float32),
                pltpu.VMEM((p_per_w,), jnp.float32),
                pltpu.SemaphoreType.DMA,
                pltpu.SemaphoreType.DMA,
                pltpu.SemaphoreType.DMA,
            ]
        ),
    )


def _dense_body(c, m, d, center_ref, ctx_ref, mu_ref, sig_ref, tct_ref,
                numo_ref, wc_ref, wx_ref, fb_ref, uw_ref, ub_ref, vw_ref,
                vb_ref, score_ref, sigq_ref):
    bb = center_ref.shape[0]
    f32 = jnp.float32
    ctx = ctx_ref[...]
    hx = lax.dot_general(ctx, wx_ref[...], (((2,), (1,)), ((), ())),
                         preferred_element_type=f32)
    hc = lax.dot_general(center_ref[...], wc_ref[...], (((1,), (1,)), ((), ())),
                         preferred_element_type=f32)
    hr = jnp.maximum(hx + hc[None, :, :] + fb_ref[...][None, :, :], 0.0)
    h = jnp.mean(hr, axis=0)
    mu_q = lax.dot_general(h, uw_ref[...], (((1,), (1,)), ((), ())),
                           preferred_element_type=f32) + ub_ref[...]
    lsq = jnp.sum(h * vw_ref[...], axis=1) + vb_ref[0, 0]
    var_q = jnp.exp(2.0 * lsq)

    norm = jnp.maximum(tct_ref[...], 1.0)
    inv_n = 1.0 / norm
    ns = mu_ref[...]
    ones_d = jnp.full((d, 1), 1.0, dtype=f32)
    q2 = lax.dot_general(mu_q * mu_q, ones_d, (((1,), (0,)), ((), ())),
                         preferred_element_type=f32)
    p2 = lax.dot_general(ns * ns, ones_d, (((2,), (0,)), ((), ())),
                         preferred_element_type=f32)[..., 0]
    t3 = ns * mu_q[:, None, :]
    cross = lax.dot_general(t3, ones_d, (((2,), (0,)), ((), ())),
                            preferred_element_type=f32)[..., 0]
    sq = q2 - 2.0 * cross * inv_n + p2 * inv_n * inv_n
    sig_p = sig_ref[...] * inv_n
    kl = (d * (jnp.log(sig_p) - lsq[:, None])
          + (d * var_q[:, None] + sq) / (2.0 * sig_p * sig_p)
          - 0.5 * d)
    mrange = lax.broadcasted_iota(jnp.int32, (bb, m), 1).astype(f32)
    score = jnp.where(mrange >= numo_ref[...], -jnp.inf, -kl)
    score_ref[...] = score
    sigq_ref[...] = jnp.exp(lsq)[:, None]


def _make_dense(b, c, m, d, bb):
    grid = (b // bb,)
    full = lambda i: (0, 0)
    return pl.pallas_call(
        functools.partial(_dense_body, c, m, d),
        grid=grid,
        in_specs=[
            pl.BlockSpec((bb, d), lambda i: (i, 0)),
            pl.BlockSpec((c, bb, d), lambda i: (0, i, 0)),
            pl.BlockSpec((bb, m, d), lambda i: (i, 0, 0)),
            pl.BlockSpec((bb, m), lambda i: (i, 0)),
            pl.BlockSpec((bb, m), lambda i: (i, 0)),
            pl.BlockSpec((bb, 1), lambda i: (i, 0)),
            pl.BlockSpec((d, d), full),
            pl.BlockSpec((d, d), full),
            pl.BlockSpec((1, d), full),
            pl.BlockSpec((d, d), full),
            pl.BlockSpec((1, d), full),
            pl.BlockSpec((1, d), full),
            pl.BlockSpec((1, 1), full),
        ],
        out_specs=[
            pl.BlockSpec((bb, m), lambda i: (i, 0)),
            pl.BlockSpec((bb, 1), lambda i: (i, 0)),
        ],
        out_shape=[
            jax.ShapeDtypeStruct((b, m), jnp.float32),
            jax.ShapeDtypeStruct((b, 1), jnp.float32),
        ],
    )


def kernel(sf_ids, context_ids, lf_ids, target_lf_ids, lf_token_ct,
           num_outputs, emb_mu, emb_log_sigma, enc_emb, f_w, f_b, u_w, u_b,
           v_w, v_b):
    b, c = context_ids.shape
    m, l = lf_ids.shape[1], lf_ids.shape[2]
    v, d = emb_mu.shape

    n_ids = b * (c + 1)
    gran = NW * CHUNK
    n_pad = -(-n_ids // gran) * gran
    all_ids = jnp.concatenate(
        [context_ids.T.reshape(-1), sf_ids,
         jnp.zeros((n_pad - n_ids,), jnp.int32)])
    ids_t = lf_ids.reshape(b * m, l).T.reshape(-1)
    mu_sum, sig_sum = _make_lf_gather(b * m, l, v, d)(
        emb_mu, emb_log_sigma.reshape(-1), ids_t)
    ctx2, center = _make_enc_gather(n_pad, b * c, b, v, d)(enc_emb, all_ids)
    ctx = ctx2.reshape(c, b, d)
    mu_sum = mu_sum.reshape(b, m, d)
    sig_sum = sig_sum.reshape(b, m)

    wc = f_w[:, :d]
    wx = f_w[:, d:]
    score, sigq = _make_dense(b, c, m, d, 128)(
        center, ctx, mu_sum, sig_sum, lf_token_ct,
        num_outputs.astype(jnp.float32)[:, None],
        wc, wx, f_b[None, :], u_w, u_b[None, :], v_w, v_b[:, None])
    return score, target_lf_ids, sigq
